# baseline (speedup 1.0000x reference)
import functools

import jax
import jax.numpy as jnp
from jax import lax
from jax.experimental import pallas as pl
from jax.experimental.pallas import tpu as pltpu
from jax.experimental.pallas import tpu_sc as plsc

_D = 64
_NW = 32
_S = 50
_B = 16384
_BPT = _B // _NW
_BLK = 128
_NBLK = _BPT // _BLK
_NT = 4
_L = 16


@functools.lru_cache(maxsize=None)
def _make_gather():
    mesh = plsc.VectorSubcoreMesh(core_axis_name="c", subcore_axis_name="s")

    @functools.partial(
        pl.kernel,
        mesh=mesh,
        compiler_params=pltpu.CompilerParams(
            use_tc_tiling_on_sc=False, needs_layout_passes=False),
        out_type=(
            jax.ShapeDtypeStruct((_S, _D, _B), jnp.float32),
            jax.ShapeDtypeStruct((_S, _D, _B), jnp.float32),
        ),
        scratch_types=[
            pltpu.VMEM((2, _BPT), jnp.int32),
            pltpu.VMEM((_NBLK, _BLK, _D), jnp.float32),
            pltpu.VMEM((_NBLK, _BLK, _D), jnp.float32),
            pltpu.VMEM((_NT, _D, _BLK), jnp.float32),
        ]
        + [pltpu.SemaphoreType.DMA] * (_NBLK + _NT),
    )
    def gather2(idx_hbm, pos_hbm, neg_hbm, out_p, out_n,
                idx_v, rows_p, rows_n, tbuf, *sems):
        gsem = sems[:_NBLK]
        csem = sems[_NBLK:]
        wid = lax.axis_index("s") * 2 + lax.axis_index("c")
        b0 = wid * _BPT

        def load_idx(s, sp):
            pltpu.sync_copy(idx_hbm.at[s, pl.ds(b0, _BPT)], idx_v.at[sp])

        def fire_gathers(sp, blk):
            isl = idx_v.at[sp, pl.ds(blk * _BLK, _BLK)]
            pltpu.async_copy(pos_hbm.at[isl], rows_p.at[blk], gsem[blk])
            pltpu.async_copy(neg_hbm.at[isl], rows_n.at[blk], gsem[blk])

        def wait_gathers(sp, blk):
            isl = idx_v.at[sp, pl.ds(blk * _BLK, _BLK)]
            pltpu.make_async_copy(pos_hbm.at[isl], rows_p.at[blk], gsem[blk]).wait()
            pltpu.make_async_copy(neg_hbm.at[isl], rows_n.at[blk], gsem[blk]).wait()

        def transpose_blk(rows, blk, tp):
            src = rows.at[blk]

            def dbody(d, carry):
                col = jnp.full((_L,), d, jnp.int32)
                for g in range(_BLK // _L):
                    ridx = lax.iota(jnp.int32, _L) + g * _L
                    v = plsc.load_gather(src, [ridx, col])
                    tbuf[tp, d, pl.ds(g * _L, _L)] = v
                return carry

            lax.fori_loop(0, _D, dbody, 0)

        def fire_copy(out, s, blk, tp):
            dst = out.at[s, :, pl.ds(b0 + blk * _BLK, _BLK)]
            pltpu.async_copy(tbuf.at[tp], dst, csem[tp])

        def wait_copy(out, s, blk, tp):
            dst = out.at[s, :, pl.ds(b0 + blk * _BLK, _BLK)]
            pltpu.make_async_copy(tbuf.at[tp], dst, csem[tp]).wait()

        def drain_tp(tp):
            dst = out_p.at[0, :, pl.ds(b0, _BLK)]
            pltpu.make_async_copy(tbuf.at[tp], dst, csem[tp]).wait()

        load_idx(0, 0)
        for blk in range(_NBLK):
            fire_gathers(0, blk)

        def body(s, carry):
            sp = lax.rem(s, 2)
            spn = lax.rem(s + 1, 2)

            @pl.when(s + 1 < _S)
            def _():
                load_idx(s + 1, spn)

            for blk in range(_NBLK):
                wait_gathers(sp, blk)
                for tbl in range(2):
                    tp = (2 * blk + tbl) % _NT
                    rows = rows_p if tbl == 0 else rows_n
                    out = out_p if tbl == 0 else out_n
                    if blk < _NT // 2:
                        @pl.when(s > 0)
                        def _():
                            drain_tp(tp)
                    else:
                        drain_tp(tp)
                    transpose_blk(rows, blk, tp)
                    fire_copy(out, s, blk, tp)

                @pl.when(s + 1 < _S)
                def _():
                    fire_gathers(spn, blk)

            return carry

        lax.fori_loop(0, _S, body, 0)
        for tp in range(_NT):
            drain_tp(tp)

    return gather2


def kernel(input, pos_table, neg_table):
    idx_t = input.T.astype(jnp.int32)
    out_p, out_n = _make_gather()(idx_t, pos_table, neg_table)
    return (out_p.transpose(2, 0, 1), out_n.transpose(2, 0, 1))


# speedup vs baseline: 1.2822x; 1.2822x over previous
import functools

import jax
import jax.numpy as jnp
from jax import lax
from jax.experimental import pallas as pl
from jax.experimental.pallas import tpu as pltpu
from jax.experimental.pallas import tpu_sc as plsc

_D = 64
_NW = 32
_S = 50
_B = 16384
_BPT = _B // _NW
_BLK = 128
_NBLK = _BPT // _BLK
_NT = 4
_L = 16


@functools.lru_cache(maxsize=None)
def _make_gather():
    mesh = plsc.VectorSubcoreMesh(core_axis_name="c", subcore_axis_name="s")

    @functools.partial(
        pl.kernel,
        mesh=mesh,
        compiler_params=pltpu.CompilerParams(
            use_tc_tiling_on_sc=False, needs_layout_passes=False),
        out_type=(
            jax.ShapeDtypeStruct((_S, _D // 8, _B // _BLK, 8, _BLK), jnp.float32),
            jax.ShapeDtypeStruct((_S, _D // 8, _B // _BLK, 8, _BLK), jnp.float32),
        ),
        scratch_types=[
            pltpu.VMEM((2, _BPT), jnp.int32),
            pltpu.VMEM((_NBLK, _BLK, _D), jnp.float32),
            pltpu.VMEM((_NBLK, _BLK, _D), jnp.float32),
            pltpu.VMEM((_NT, _D, _BLK), jnp.float32),
        ]
        + [pltpu.SemaphoreType.DMA] * (_NBLK + _NT),
    )
    def gather2(idx_hbm, pos_hbm, neg_hbm, out_p, out_n,
                idx_v, rows_p, rows_n, tbuf, *sems):
        gsem = sems[:_NBLK]
        csem = sems[_NBLK:]
        wid = lax.axis_index("s") * 2 + lax.axis_index("c")
        b0 = wid * _BPT

        def load_idx(s, sp):
            pltpu.sync_copy(idx_hbm.at[s, pl.ds(b0, _BPT)], idx_v.at[sp])

        def fire_gathers(sp, blk):
            isl = idx_v.at[sp, pl.ds(blk * _BLK, _BLK)]
            pltpu.async_copy(pos_hbm.at[isl], rows_p.at[blk], gsem[blk])
            pltpu.async_copy(neg_hbm.at[isl], rows_n.at[blk], gsem[blk])

        def wait_gathers(sp, blk):
            isl = idx_v.at[sp, pl.ds(blk * _BLK, _BLK)]
            pltpu.make_async_copy(pos_hbm.at[isl], rows_p.at[blk], gsem[blk]).wait()
            pltpu.make_async_copy(neg_hbm.at[isl], rows_n.at[blk], gsem[blk]).wait()

        def transpose_blk(rows, blk, tp):
            tb = tbuf.at[tp]
            ridx = [lax.iota(jnp.int32, _L) + g * _L for g in range(_D // _L)]

            def jbody(j, carry):
                col = jnp.full((_L,), j, jnp.int32)
                for g in range(_D // _L):
                    v = rows[blk, j, pl.ds(g * _L, _L)]
                    plsc.store_scatter(tb, [ridx[g], col], v)
                return carry

            lax.fori_loop(0, _BLK, jbody, 0, unroll=2)

        def fire_tiles(out, s, blk, tp):
            bt = _NBLK * wid + blk
            for dt in range(_D // 8):
                src = tbuf.at[tp, pl.ds(dt * 8, 8)]
                pltpu.async_copy(src, out.at[s, dt, bt], csem[tp])

        def drain_tiles(tp):
            for dt in range(_D // 8):
                src = tbuf.at[tp, pl.ds(dt * 8, 8)]
                pltpu.make_async_copy(src, out_p.at[0, dt, 0], csem[tp]).wait()

        load_idx(0, 0)
        for blk in range(_NBLK):
            fire_gathers(0, blk)

        def body(s, carry):
            sp = lax.rem(s, 2)
            spn = lax.rem(s + 1, 2)

            @pl.when(s + 1 < _S)
            def _():
                load_idx(s + 1, spn)

            for blk in range(_NBLK):
                wait_gathers(sp, blk)
                for tbl in range(2):
                    tp = (2 * blk + tbl) % _NT
                    rows = rows_p if tbl == 0 else rows_n
                    out = out_p if tbl == 0 else out_n
                    if blk < _NT // 2:
                        @pl.when(s > 0)
                        def _():
                            drain_tiles(tp)
                    else:
                        drain_tiles(tp)
                    transpose_blk(rows, blk, tp)
                    fire_tiles(out, s, blk, tp)

                @pl.when(s + 1 < _S)
                def _():
                    fire_gathers(spn, blk)

            return carry

        lax.fori_loop(0, _S, body, 0)
        for tp in range(_NT):
            drain_tiles(tp)

    return gather2


def kernel(input, pos_table, neg_table):
    idx_t = input.T.astype(jnp.int32)
    out_p, out_n = _make_gather()(idx_t, pos_table, neg_table)

    def detile(o):
        return o.transpose(2, 4, 0, 1, 3).reshape(_B, _S, _D)

    return (detile(out_p), detile(out_n))


# speedup vs baseline: 1.5734x; 1.2271x over previous
import functools

import jax
import jax.numpy as jnp
from jax import lax
from jax.experimental import pallas as pl
from jax.experimental.pallas import tpu as pltpu
from jax.experimental.pallas import tpu_sc as plsc

_D = 64
_NW = 32
_S = 50
_B = 16384
_BPT = _B // _NW
_BLK = 128
_NBLK = _BPT // _BLK
_NT = 4
_L = 16


@functools.lru_cache(maxsize=None)
def _make_gather():
    mesh = plsc.VectorSubcoreMesh(core_axis_name="c", subcore_axis_name="s")

    @functools.partial(
        pl.kernel,
        mesh=mesh,
        compiler_params=pltpu.CompilerParams(
            use_tc_tiling_on_sc=False, needs_layout_passes=False),
        out_type=(
            jax.ShapeDtypeStruct((_S, _D // 8, _B // _BLK, 8, _BLK), jnp.float32),
            jax.ShapeDtypeStruct((_S, _D // 8, _B // _BLK, 8, _BLK), jnp.float32),
        ),
        scratch_types=[
            pltpu.VMEM((2, _BPT), jnp.int32),
            pltpu.VMEM((_NBLK, _BLK, _D), jnp.float32),
            pltpu.VMEM((_NBLK, _BLK, _D), jnp.float32),
            pltpu.VMEM((_NT, _D, _BLK), jnp.float32),
        ]
        + [pltpu.SemaphoreType.DMA] * (_NBLK + _NT),
    )
    def gather2(idx_hbm, pos_hbm, neg_hbm, out_p, out_n,
                idx_v, rows_p, rows_n, tbuf, *sems):
        gsem = sems[:_NBLK]
        csem = sems[_NBLK:]
        wid = lax.axis_index("s") * 2 + lax.axis_index("c")
        b0 = wid * _BPT

        def load_idx(s, sp):
            pltpu.sync_copy(idx_hbm.at[s, pl.ds(b0, _BPT)], idx_v.at[sp])

        def fire_gathers(sp, blk):
            isl = idx_v.at[sp, pl.ds(blk * _BLK, _BLK)]
            pltpu.async_copy(pos_hbm.at[isl], rows_p.at[blk], gsem[blk])
            pltpu.async_copy(neg_hbm.at[isl], rows_n.at[blk], gsem[blk])

        def wait_gathers(sp, blk):
            isl = idx_v.at[sp, pl.ds(blk * _BLK, _BLK)]
            pltpu.make_async_copy(pos_hbm.at[isl], rows_p.at[blk], gsem[blk]).wait()
            pltpu.make_async_copy(neg_hbm.at[isl], rows_n.at[blk], gsem[blk]).wait()

        def transpose_blk(rows, blk, tp):
            tb = tbuf.at[tp]
            ridx = [lax.iota(jnp.int32, _L) + g * _L for g in range(_D // _L)]

            @plsc.parallel_loop(0, _BLK, 1, unroll=8)
            def _(j):
                col = jnp.full((_L,), j, jnp.int32)
                for g in range(_D // _L):
                    v = rows[blk, j, pl.ds(g * _L, _L)]
                    plsc.store_scatter(tb, [ridx[g], col], v)

        def fire_tiles(out, s, blk, tp):
            bt = _NBLK * wid + blk
            for dt in range(_D // 8):
                src = tbuf.at[tp, pl.ds(dt * 8, 8)]
                pltpu.async_copy(src, out.at[s, dt, bt], csem[tp])

        def drain_tiles(tp):
            for dt in range(_D // 8):
                src = tbuf.at[tp, pl.ds(dt * 8, 8)]
                pltpu.make_async_copy(src, out_p.at[0, dt, 0], csem[tp]).wait()

        load_idx(0, 0)
        for blk in range(_NBLK):
            fire_gathers(0, blk)

        def body(s, carry):
            sp = lax.rem(s, 2)
            spn = lax.rem(s + 1, 2)

            @pl.when(s + 1 < _S)
            def _():
                load_idx(s + 1, spn)

            for blk in range(_NBLK):
                wait_gathers(sp, blk)
                for tbl in range(2):
                    tp = (2 * blk + tbl) % _NT
                    rows = rows_p if tbl == 0 else rows_n
                    out = out_p if tbl == 0 else out_n
                    if blk < _NT // 2:
                        @pl.when(s > 0)
                        def _():
                            drain_tiles(tp)
                    else:
                        drain_tiles(tp)
                    transpose_blk(rows, blk, tp)
                    fire_tiles(out, s, blk, tp)

                @pl.when(s + 1 < _S)
                def _():
                    fire_gathers(spn, blk)

            return carry

        lax.fori_loop(0, _S, body, 0)
        for tp in range(_NT):
            drain_tiles(tp)

    return gather2


def kernel(input, pos_table, neg_table):
    idx_t = input.T.astype(jnp.int32)
    out_p, out_n = _make_gather()(idx_t, pos_table, neg_table)

    def detile(o):
        return o.transpose(2, 4, 0, 1, 3).reshape(_B, _S, _D)

    return (detile(out_p), detile(out_n))


# speedup vs baseline: 2.8679x; 1.8228x over previous
import functools

import jax
import jax.numpy as jnp
from jax import lax
from jax.experimental import pallas as pl
from jax.experimental.pallas import tpu as pltpu
from jax.experimental.pallas import tpu_sc as plsc

_D = 64
_NW = 32
_S = 50
_B = 16384
_BPT = _B // _NW
_BLK = 128
_NBLK = _BPT // _BLK
_NT = 4
_L = 16
_TP = 129


@functools.lru_cache(maxsize=None)
def _make_gather():
    mesh = plsc.VectorSubcoreMesh(core_axis_name="c", subcore_axis_name="s")

    @functools.partial(
        pl.kernel,
        mesh=mesh,
        compiler_params=pltpu.CompilerParams(
            use_tc_tiling_on_sc=False, needs_layout_passes=False),
        out_type=(
            jax.ShapeDtypeStruct((_S, _D // 8, _B // _BLK, 8, _BLK), jnp.float32),
            jax.ShapeDtypeStruct((_S, _D // 8, _B // _BLK, 8, _BLK), jnp.float32),
        ),
        scratch_types=[
            pltpu.VMEM((2, _BPT), jnp.int32),
            pltpu.VMEM((_NBLK, _BLK, _D), jnp.float32),
            pltpu.VMEM((_NBLK, _BLK, _D), jnp.float32),
            pltpu.VMEM((_NT, _D, _TP), jnp.float32),
        ]
        + [pltpu.SemaphoreType.DMA] * (_NBLK + _NT),
    )
    def gather2(idx_hbm, pos_hbm, neg_hbm, out_p, out_n,
                idx_v, rows_p, rows_n, tbuf, *sems):
        gsem = sems[:_NBLK]
        csem = sems[_NBLK:]
        wid = lax.axis_index("s") * 2 + lax.axis_index("c")
        b0 = wid * _BPT

        def load_idx(s, sp):
            pltpu.sync_copy(idx_hbm.at[s, pl.ds(b0, _BPT)], idx_v.at[sp])

        def fire_gathers(sp, blk):
            isl = idx_v.at[sp, pl.ds(blk * _BLK, _BLK)]
            pltpu.async_copy(pos_hbm.at[isl], rows_p.at[blk], gsem[blk])
            pltpu.async_copy(neg_hbm.at[isl], rows_n.at[blk], gsem[blk])

        def wait_gathers(sp, blk):
            isl = idx_v.at[sp, pl.ds(blk * _BLK, _BLK)]
            pltpu.make_async_copy(pos_hbm.at[isl], rows_p.at[blk], gsem[blk]).wait()
            pltpu.make_async_copy(neg_hbm.at[isl], rows_n.at[blk], gsem[blk]).wait()

        def transpose_blk(rows, blk, tp):
            tb = tbuf.at[tp]
            ridx = [lax.iota(jnp.int32, _L) + g * _L for g in range(_D // _L)]

            @plsc.parallel_loop(0, _BLK, 1, unroll=8)
            def _(j):
                col = jnp.full((_L,), j, jnp.int32)
                for g in range(_D // _L):
                    v = rows[blk, j, pl.ds(g * _L, _L)]
                    plsc.store_scatter(tb, [ridx[g], col], v)

        def fire_tiles(out, s, blk, tp):
            bt = _NBLK * wid + blk
            for dt in range(_D // 8):
                src = tbuf.at[tp, pl.ds(dt * 8, 8), pl.ds(0, _BLK)]
                pltpu.async_copy(src, out.at[s, dt, bt], csem[tp])

        def drain_tiles(tp):
            for dt in range(_D // 8):
                src = tbuf.at[tp, pl.ds(dt * 8, 8), pl.ds(0, _BLK)]
                pltpu.make_async_copy(src, out_p.at[0, dt, 0], csem[tp]).wait()

        load_idx(0, 0)
        for blk in range(_NBLK):
            fire_gathers(0, blk)

        def body(s, carry):
            sp = lax.rem(s, 2)
            spn = lax.rem(s + 1, 2)

            @pl.when(s + 1 < _S)
            def _():
                load_idx(s + 1, spn)

            for blk in range(_NBLK):
                wait_gathers(sp, blk)
                for tbl in range(2):
                    tp = (2 * blk + tbl) % _NT
                    rows = rows_p if tbl == 0 else rows_n
                    out = out_p if tbl == 0 else out_n
                    if blk < _NT // 2:
                        @pl.when(s > 0)
                        def _():
                            drain_tiles(tp)
                    else:
                        drain_tiles(tp)
                    transpose_blk(rows, blk, tp)
                    fire_tiles(out, s, blk, tp)

                @pl.when(s + 1 < _S)
                def _():
                    fire_gathers(spn, blk)

            return carry

        lax.fori_loop(0, _S, body, 0)
        for tp in range(_NT):
            drain_tiles(tp)

    return gather2


def kernel(input, pos_table, neg_table):
    idx_t = input.T.astype(jnp.int32)
    out_p, out_n = _make_gather()(idx_t, pos_table, neg_table)

    def detile(o):
        return o.transpose(2, 4, 0, 1, 3).reshape(_B, _S, _D)

    return (detile(out_p), detile(out_n))


# speedup vs baseline: 3.3135x; 1.1554x over previous
import functools

import jax
import jax.numpy as jnp
from jax import lax
from jax.experimental import pallas as pl
from jax.experimental.pallas import tpu as pltpu
from jax.experimental.pallas import tpu_sc as plsc

_D = 64
_NW = 32
_S = 50
_B = 16384
_BPT = _B // _NW
_BLK = 128
_NBLK = _BPT // _BLK
_NT = 4
_L = 16
_TP = 129


@functools.lru_cache(maxsize=None)
def _make_gather():
    mesh = plsc.VectorSubcoreMesh(core_axis_name="c", subcore_axis_name="s")

    @functools.partial(
        pl.kernel,
        mesh=mesh,
        compiler_params=pltpu.CompilerParams(
            use_tc_tiling_on_sc=False, needs_layout_passes=False),
        out_type=(
            jax.ShapeDtypeStruct((_S, _D // 8, _B // _BLK, 8, _BLK), jnp.float32),
            jax.ShapeDtypeStruct((_S, _D // 8, _B // _BLK, 8, _BLK), jnp.float32),
        ),
        scratch_types=[
            pltpu.VMEM((2, _BPT), jnp.int32),
            pltpu.VMEM((_NBLK, _BLK, 2 * _D), jnp.float32),
            pltpu.VMEM((_NT, _D, _TP), jnp.float32),
        ]
        + [pltpu.SemaphoreType.DMA] * (_NBLK + _NT),
    )
    def gather2(idx_hbm, tab_hbm, out_p, out_n, idx_v, rows, tbuf, *sems):
        gsem = sems[:_NBLK]
        csem = sems[_NBLK:]
        wid = lax.axis_index("s") * 2 + lax.axis_index("c")
        b0 = wid * _BPT

        def load_idx(s, sp):
            pltpu.sync_copy(idx_hbm.at[s, pl.ds(b0, _BPT)], idx_v.at[sp])

        def fire_gathers(sp, blk):
            isl = idx_v.at[sp, pl.ds(blk * _BLK, _BLK)]
            pltpu.async_copy(tab_hbm.at[isl], rows.at[blk], gsem[blk])

        def wait_gathers(sp, blk):
            isl = idx_v.at[sp, pl.ds(blk * _BLK, _BLK)]
            pltpu.make_async_copy(tab_hbm.at[isl], rows.at[blk], gsem[blk]).wait()

        def transpose_blk(blk, tbl, tp):
            tb = tbuf.at[tp]
            ridx = [lax.iota(jnp.int32, _L) + g * _L for g in range(_D // _L)]

            @plsc.parallel_loop(0, _BLK, 1, unroll=8)
            def _(j):
                col = jnp.full((_L,), j, jnp.int32)
                for g in range(_D // _L):
                    v = rows[blk, j, pl.ds(tbl * _D + g * _L, _L)]
                    plsc.store_scatter(tb, [ridx[g], col], v)

        def fire_tiles(out, s, blk, tp):
            bt = _NBLK * wid + blk
            for dt in range(_D // 8):
                src = tbuf.at[tp, pl.ds(dt * 8, 8), pl.ds(0, _BLK)]
                pltpu.async_copy(src, out.at[s, dt, bt], csem[tp])

        def drain_tiles(tp):
            for dt in range(_D // 8):
                src = tbuf.at[tp, pl.ds(dt * 8, 8), pl.ds(0, _BLK)]
                pltpu.make_async_copy(src, out_p.at[0, dt, 0], csem[tp]).wait()

        load_idx(0, 0)
        for blk in range(_NBLK):
            fire_gathers(0, blk)

        def body(s, carry):
            sp = lax.rem(s, 2)
            spn = lax.rem(s + 1, 2)

            @pl.when(s + 1 < _S)
            def _():
                load_idx(s + 1, spn)

            for blk in range(_NBLK):
                wait_gathers(sp, blk)
                for tbl in range(2):
                    tp = (2 * blk + tbl) % _NT
                    out = out_p if tbl == 0 else out_n
                    if blk < _NT // 2:
                        @pl.when(s > 0)
                        def _():
                            drain_tiles(tp)
                    else:
                        drain_tiles(tp)
                    transpose_blk(blk, tbl, tp)
                    fire_tiles(out, s, blk, tp)

                @pl.when(s + 1 < _S)
                def _():
                    fire_gathers(spn, blk)

            return carry

        lax.fori_loop(0, _S, body, 0)
        for tp in range(_NT):
            drain_tiles(tp)

    return gather2


def kernel(input, pos_table, neg_table):
    idx_t = input.T.astype(jnp.int32)
    tab = jnp.concatenate([pos_table, neg_table], axis=1)
    out_p, out_n = _make_gather()(idx_t, tab)

    def detile(o):
        return o.transpose(2, 4, 0, 1, 3).reshape(_B, _S, _D)

    return (detile(out_p), detile(out_n))
